# baseline (speedup 1.0000x reference)
import functools

import jax
import jax.numpy as jnp
from jax.experimental import pallas as pl
from jax.experimental.pallas import tpu as pltpu
from jax.experimental.pallas import tpu_sc as plsc

_B = 4096
_F = 40960
_FSC = 4096
_FC = _F - _FSC
_BB = 64
_NI = _B // _BB

_NTEC = 32
_RPT = _B // _NTEC
_RG = 8
_NG = _RPT // _RG
_LANES = 16


def _dot8(x, w):
    return jax.lax.dot_general(x, w, (((1,), (0,)), ((), ())),
                               precision=jax.lax.Precision.DEFAULT,
                               preferred_element_type=jnp.float32)


def _tc_main_body(wf_ref, bf_ref, ftwT_ref, accw_ref, accb_ref):
    ftw = ftwT_ref[...]
    accw_ref[...] = _dot8(wf_ref[...], ftw)
    accb_ref[...] = _dot8(bf_ref[...], ftw)


def _tc_main(white, black, ftwT):
    return pl.pallas_call(
        _tc_main_body,
        grid=(_NI,),
        in_specs=[
            pl.BlockSpec((_BB, _FC), lambda i: (i, 0)),
            pl.BlockSpec((_BB, _FC), lambda i: (i, 0)),
            pl.BlockSpec((_FC, 8), lambda i: (0, 0)),
        ],
        out_specs=[
            pl.BlockSpec((_BB, 8), lambda i: (i, 0)),
            pl.BlockSpec((_BB, 8), lambda i: (i, 0)),
        ],
        out_shape=[
            jax.ShapeDtypeStruct((_B, 8), jnp.float32),
            jax.ShapeDtypeStruct((_B, 8), jnp.float32),
        ],
        compiler_params=pltpu.CompilerParams(
            dimension_semantics=("arbitrary",),
        ),
    )(white, black, ftwT)


def _sc_partial(white, black, wsc):
    mesh = plsc.VectorSubcoreMesh(core_axis_name="c", subcore_axis_name="s")

    @functools.partial(
        pl.kernel,
        out_type=jax.ShapeDtypeStruct((_B, 2 * 4 * _LANES), jnp.float32),
        mesh=mesh,
        scratch_types=[
            pltpu.VMEM((4, _FSC), jnp.float32),
            pltpu.VMEM((_RG, _FSC), jnp.float32),
            pltpu.VMEM((_RG, _FSC), jnp.float32),
            pltpu.VMEM((_RG, 2 * 4 * _LANES), jnp.float32),
            pltpu.SemaphoreType.DMA((4,)),
        ],
    )
    def sc_kernel(wf_hbm, bf_hbm, wsc_hbm, o_hbm, wbuf, xw, xb, obuf, sem):
        c = jax.lax.axis_index("c")
        s = jax.lax.axis_index("s")
        r0 = (c * 16 + s) * _RPT

        pltpu.make_async_copy(wsc_hbm, wbuf, sem.at[0]).start()
        pltpu.make_async_copy(wsc_hbm, wbuf, sem.at[0]).wait()

        def _xsrc(src, g):
            return src.at[pl.ds(r0 + g * _RG, _RG), pl.ds(_FC, _FSC)]

        pltpu.make_async_copy(_xsrc(wf_hbm, 0), xw, sem.at[0]).start()
        pltpu.make_async_copy(_xsrc(bf_hbm, 0), xb, sem.at[1]).start()

        zero = jnp.zeros((_LANES,), jnp.float32)
        wrows = tuple(wbuf.at[j] for j in range(4))

        def _consume(xbuf, g, st, semid):
            pltpu.make_async_copy(_xsrc((wf_hbm, bf_hbm)[st], g), xbuf,
                                  sem.at[semid]).wait()
            for p in range(_RG // 2):
                ra = xbuf.at[2 * p]
                rb = xbuf.at[2 * p + 1]

                def _cbody(ci, carry):
                    a0, a1, a2, a3, b0, b1, b2, b3 = carry
                    off = ci * _LANES
                    xa = ra[pl.ds(off, _LANES)]
                    xv = rb[pl.ds(off, _LANES)]
                    w0 = wrows[0][pl.ds(off, _LANES)]
                    w1 = wrows[1][pl.ds(off, _LANES)]
                    w2 = wrows[2][pl.ds(off, _LANES)]
                    w3 = wrows[3][pl.ds(off, _LANES)]
                    return (a0 + xa * w0, a1 + xa * w1, a2 + xa * w2,
                            a3 + xa * w3, b0 + xv * w0, b1 + xv * w1,
                            b2 + xv * w2, b3 + xv * w3)

                accs = jax.lax.fori_loop(
                    0, _FSC // _LANES, _cbody, (zero,) * 8, unroll=2)
                base = st * 4 * _LANES
                for j in range(4):
                    obuf.at[2 * p][pl.ds(base + j * _LANES, _LANES)] = accs[j]
                    obuf.at[2 * p + 1][pl.ds(base + j * _LANES, _LANES)] = (
                        accs[4 + j])

            @pl.when(g < _NG - 1)
            def _():
                pltpu.make_async_copy(_xsrc((wf_hbm, bf_hbm)[st], g + 1),
                                      xbuf, sem.at[semid]).start()

        def _gbody(g, carry):
            @pl.when(g > 0)
            def _():
                pltpu.make_async_copy(
                    obuf, o_hbm.at[pl.ds(r0 + (g - 1) * _RG, _RG), :],
                    sem.at[2]).wait()
            _consume(xw, g, 0, 0)
            _consume(xb, g, 1, 1)
            pltpu.make_async_copy(
                obuf, o_hbm.at[pl.ds(r0 + g * _RG, _RG), :],
                sem.at[2]).start()
            return carry

        jax.lax.fori_loop(0, _NG, _gbody, 0)
        pltpu.make_async_copy(
            obuf, o_hbm.at[pl.ds(r0 + (_NG - 1) * _RG, _RG), :],
            sem.at[2]).wait()

    return sc_kernel(white, black, wsc)


def _tc_combine_body(accw_ref, accb_ref, sc_ref, gw_ref, gb_ref, stm_ref,
                     ftb_ref, l1aT_ref, l1bT_ref, l1b_ref, l2wT_ref, l2b_ref,
                     out_ref):
    scp = sc_ref[...]
    ftb = ftb_ref[...]
    w8 = accw_ref[...] + _dot8(scp, gw_ref[...]) + ftb
    b8 = accb_ref[...] + _dot8(scp, gb_ref[...]) + ftb
    stm = stm_ref[...]
    mix1 = b8 + stm * (w8 - b8)
    mix2 = w8 + stm * (b8 - w8)
    c1 = jnp.clip(mix1, 0.0, 1.0)
    c2 = jnp.clip(mix2, 0.0, 1.0)
    h = jnp.dot(c1, l1aT_ref[...], preferred_element_type=jnp.float32)
    h += jnp.dot(c2, l1bT_ref[...], preferred_element_type=jnp.float32)
    h = jnp.clip(h + l1b_ref[...], 0.0, 1.0)
    out_ref[...] = jnp.dot(h, l2wT_ref[...],
                           preferred_element_type=jnp.float32) + l2b_ref[...]


@functools.partial(jax.jit, static_argnames=())
def kernel(white_features, black_features, stm, ft_w, ft_b, l1_w, l1_b, l2_w,
           l2_b):
    f32 = jnp.float32
    ftwT = jnp.pad(ft_w[:, :_FC], ((0, 4), (0, 0))).T.astype(jnp.bfloat16)
    wsc = ft_w[:, _FC:]
    ftb8 = jnp.pad(ft_b, (0, 4)).reshape(1, 8)
    l1aT = jnp.pad(l1_w[:, :4].T, ((0, 4), (0, 0)))
    l1bT = jnp.pad(l1_w[:, 4:].T, ((0, 4), (0, 0)))
    l1b2 = l1_b.reshape(1, 8)
    l2wT = l2_w.T
    l2b2 = l2_b.reshape(1, 1)
    stm2 = stm.reshape(_B, 1)
    lane = jnp.arange(128)
    col = jnp.arange(8)
    gw = ((lane[:, None] < 64) & (lane[:, None] // 16 == col[None, :])
          ).astype(f32)
    gb = ((lane[:, None] >= 64) & ((lane[:, None] - 64) // 16 == col[None, :])
          ).astype(f32)

    accw, accb = _tc_main(white_features, black_features, ftwT)
    scp = _sc_partial(white_features, black_features, wsc)

    out = pl.pallas_call(
        _tc_combine_body,
        out_shape=jax.ShapeDtypeStruct((_B, 1), f32),
    )(accw, accb, scp, gw, gb, stm2, ftb8, l1aT, l1bT, l1b2, l2wT, l2b2)
    return out


# speedup vs baseline: 1.0034x; 1.0034x over previous
import functools

import jax
import jax.numpy as jnp
from jax.experimental import pallas as pl
from jax.experimental.pallas import tpu as pltpu
from jax.experimental.pallas import tpu_sc as plsc

_B = 4096
_F = 40960
_FSC = 2048
_FC = _F - _FSC
_BB = 64
_NI = _B // _BB

_NTEC = 32
_RPT = _B // _NTEC
_RG = 8
_NG = _RPT // _RG
_LANES = 16


def _dot8(x, w):
    return jax.lax.dot_general(x, w, (((1,), (0,)), ((), ())),
                               precision=jax.lax.Precision.DEFAULT,
                               preferred_element_type=jnp.float32)


def _tc_main_body(wf_ref, bf_ref, ftwT_ref, accw_ref, accb_ref):
    ftw = ftwT_ref[...]
    accw_ref[...] = _dot8(wf_ref[...], ftw)
    accb_ref[...] = _dot8(bf_ref[...], ftw)


def _tc_main(white, black, ftwT):
    return pl.pallas_call(
        _tc_main_body,
        grid=(_NI,),
        in_specs=[
            pl.BlockSpec((_BB, _FC), lambda i: (i, 0)),
            pl.BlockSpec((_BB, _FC), lambda i: (i, 0)),
            pl.BlockSpec((_FC, 8), lambda i: (0, 0)),
        ],
        out_specs=[
            pl.BlockSpec((_BB, 8), lambda i: (i, 0)),
            pl.BlockSpec((_BB, 8), lambda i: (i, 0)),
        ],
        out_shape=[
            jax.ShapeDtypeStruct((_B, 8), jnp.float32),
            jax.ShapeDtypeStruct((_B, 8), jnp.float32),
        ],
        compiler_params=pltpu.CompilerParams(
            dimension_semantics=("arbitrary",),
        ),
    )(white, black, ftwT)


def _sc_partial(white, black, wsc):
    mesh = plsc.VectorSubcoreMesh(core_axis_name="c", subcore_axis_name="s")

    @functools.partial(
        pl.kernel,
        out_type=jax.ShapeDtypeStruct((_B, 2 * 4 * _LANES), jnp.float32),
        mesh=mesh,
        scratch_types=[
            pltpu.VMEM((4, _FSC), jnp.float32),
            pltpu.VMEM((_RG, _FSC), jnp.float32),
            pltpu.VMEM((_RG, _FSC), jnp.float32),
            pltpu.VMEM((_RG, 2 * 4 * _LANES), jnp.float32),
            pltpu.SemaphoreType.DMA((4,)),
        ],
    )
    def sc_kernel(wf_hbm, bf_hbm, wsc_hbm, o_hbm, wbuf, xw, xb, obuf, sem):
        c = jax.lax.axis_index("c")
        s = jax.lax.axis_index("s")
        r0 = (c * 16 + s) * _RPT

        pltpu.make_async_copy(wsc_hbm, wbuf, sem.at[0]).start()
        pltpu.make_async_copy(wsc_hbm, wbuf, sem.at[0]).wait()

        def _xsrc(src, g):
            return src.at[pl.ds(r0 + g * _RG, _RG), pl.ds(_FC, _FSC)]

        pltpu.make_async_copy(_xsrc(wf_hbm, 0), xw, sem.at[0]).start()
        pltpu.make_async_copy(_xsrc(bf_hbm, 0), xb, sem.at[1]).start()

        zero = jnp.zeros((_LANES,), jnp.float32)
        wrows = tuple(wbuf.at[j] for j in range(4))

        def _consume(xbuf, g, st, semid):
            pltpu.make_async_copy(_xsrc((wf_hbm, bf_hbm)[st], g), xbuf,
                                  sem.at[semid]).wait()
            for p in range(_RG // 2):
                ra = xbuf.at[2 * p]
                rb = xbuf.at[2 * p + 1]

                def _cbody(ci, carry):
                    a0, a1, a2, a3, b0, b1, b2, b3 = carry
                    off = ci * _LANES
                    xa = ra[pl.ds(off, _LANES)]
                    xv = rb[pl.ds(off, _LANES)]
                    w0 = wrows[0][pl.ds(off, _LANES)]
                    w1 = wrows[1][pl.ds(off, _LANES)]
                    w2 = wrows[2][pl.ds(off, _LANES)]
                    w3 = wrows[3][pl.ds(off, _LANES)]
                    return (a0 + xa * w0, a1 + xa * w1, a2 + xa * w2,
                            a3 + xa * w3, b0 + xv * w0, b1 + xv * w1,
                            b2 + xv * w2, b3 + xv * w3)

                accs = jax.lax.fori_loop(
                    0, _FSC // _LANES, _cbody, (zero,) * 8, unroll=2)
                base = st * 4 * _LANES
                for j in range(4):
                    obuf.at[2 * p][pl.ds(base + j * _LANES, _LANES)] = accs[j]
                    obuf.at[2 * p + 1][pl.ds(base + j * _LANES, _LANES)] = (
                        accs[4 + j])

            @pl.when(g < _NG - 1)
            def _():
                pltpu.make_async_copy(_xsrc((wf_hbm, bf_hbm)[st], g + 1),
                                      xbuf, sem.at[semid]).start()

        def _gbody(g, carry):
            @pl.when(g > 0)
            def _():
                pltpu.make_async_copy(
                    obuf, o_hbm.at[pl.ds(r0 + (g - 1) * _RG, _RG), :],
                    sem.at[2]).wait()
            _consume(xw, g, 0, 0)
            _consume(xb, g, 1, 1)
            pltpu.make_async_copy(
                obuf, o_hbm.at[pl.ds(r0 + g * _RG, _RG), :],
                sem.at[2]).start()
            return carry

        jax.lax.fori_loop(0, _NG, _gbody, 0)
        pltpu.make_async_copy(
            obuf, o_hbm.at[pl.ds(r0 + (_NG - 1) * _RG, _RG), :],
            sem.at[2]).wait()

    return sc_kernel(white, black, wsc)


def _tc_combine_body(accw_ref, accb_ref, sc_ref, gw_ref, gb_ref, stm_ref,
                     ftb_ref, l1aT_ref, l1bT_ref, l1b_ref, l2wT_ref, l2b_ref,
                     out_ref):
    scp = sc_ref[...]
    ftb = ftb_ref[...]
    w8 = accw_ref[...] + _dot8(scp, gw_ref[...]) + ftb
    b8 = accb_ref[...] + _dot8(scp, gb_ref[...]) + ftb
    stm = stm_ref[...]
    mix1 = b8 + stm * (w8 - b8)
    mix2 = w8 + stm * (b8 - w8)
    c1 = jnp.clip(mix1, 0.0, 1.0)
    c2 = jnp.clip(mix2, 0.0, 1.0)
    h = jnp.dot(c1, l1aT_ref[...], preferred_element_type=jnp.float32)
    h += jnp.dot(c2, l1bT_ref[...], preferred_element_type=jnp.float32)
    h = jnp.clip(h + l1b_ref[...], 0.0, 1.0)
    out_ref[...] = jnp.dot(h, l2wT_ref[...],
                           preferred_element_type=jnp.float32) + l2b_ref[...]


@functools.partial(jax.jit, static_argnames=())
def kernel(white_features, black_features, stm, ft_w, ft_b, l1_w, l1_b, l2_w,
           l2_b):
    f32 = jnp.float32
    ftwT = jnp.pad(ft_w[:, :_FC], ((0, 4), (0, 0))).T.astype(jnp.bfloat16)
    wsc = ft_w[:, _FC:]
    ftb8 = jnp.pad(ft_b, (0, 4)).reshape(1, 8)
    l1aT = jnp.pad(l1_w[:, :4].T, ((0, 4), (0, 0)))
    l1bT = jnp.pad(l1_w[:, 4:].T, ((0, 4), (0, 0)))
    l1b2 = l1_b.reshape(1, 8)
    l2wT = l2_w.T
    l2b2 = l2_b.reshape(1, 1)
    stm2 = stm.reshape(_B, 1)
    lane = jnp.arange(128)
    col = jnp.arange(8)
    gw = ((lane[:, None] < 64) & (lane[:, None] // 16 == col[None, :])
          ).astype(f32)
    gb = ((lane[:, None] >= 64) & ((lane[:, None] - 64) // 16 == col[None, :])
          ).astype(f32)

    accw, accb = _tc_main(white_features, black_features, ftwT)
    scp = _sc_partial(white_features, black_features, wsc)

    out = pl.pallas_call(
        _tc_combine_body,
        out_shape=jax.ShapeDtypeStruct((_B, 1), f32),
    )(accw, accb, scp, gw, gb, stm2, ftb8, l1aT, l1bT, l1b2, l2wT, l2b2)
    return out


# speedup vs baseline: 1.0436x; 1.0401x over previous
import functools

import jax
import jax.numpy as jnp
from jax.experimental import pallas as pl
from jax.experimental.pallas import tpu as pltpu

_B = 4096
_F = 40960
_BB = 64
_NI = _B // _BB


def _dot8(x, w):
    return jax.lax.dot_general(x, w, (((1,), (0,)), ((), ())),
                               precision=jax.lax.Precision.DEFAULT,
                               preferred_element_type=jnp.float32)


def _nnue_body(stm_ref, ftb_ref, l1aT_ref, l1bT_ref, l1b_ref, l2wT_ref,
               l2b_ref, wf_ref, bf_ref, ftwT_ref, out_ref):
    ftw = ftwT_ref[...]
    ftb = ftb_ref[...]
    w8 = _dot8(wf_ref[...], ftw) + ftb
    b8 = _dot8(bf_ref[...], ftw) + ftb
    stm = stm_ref[...]
    mix1 = b8 + stm * (w8 - b8)
    mix2 = w8 + stm * (b8 - w8)
    c1 = jnp.clip(mix1, 0.0, 1.0)
    c2 = jnp.clip(mix2, 0.0, 1.0)
    h = jnp.dot(c1, l1aT_ref[...], preferred_element_type=jnp.float32)
    h += jnp.dot(c2, l1bT_ref[...], preferred_element_type=jnp.float32)
    h = jnp.clip(h + l1b_ref[...], 0.0, 1.0)
    out_ref[...] = jnp.dot(h, l2wT_ref[...],
                           preferred_element_type=jnp.float32) + l2b_ref[...]


@functools.partial(jax.jit, static_argnames=("interpret",))
def kernel(white_features, black_features, stm, ft_w, ft_b, l1_w, l1_b, l2_w,
           l2_b, interpret=False):
    f32 = jnp.float32
    ftwT = jnp.pad(ft_w, ((0, 4), (0, 0))).T.astype(jnp.bfloat16)
    ftb8 = jnp.pad(ft_b, (0, 4)).reshape(1, 8)
    l1aT = jnp.pad(l1_w[:, :4].T, ((0, 4), (0, 0)))
    l1bT = jnp.pad(l1_w[:, 4:].T, ((0, 4), (0, 0)))
    l1b2 = l1_b.reshape(1, 8)
    l2wT = l2_w.T
    l2b2 = l2_b.reshape(1, 1)
    stm2 = stm.reshape(_B, 1)

    out = pl.pallas_call(
        _nnue_body,
        grid=(_NI,),
        in_specs=[
            pl.BlockSpec((_BB, 1), lambda i: (i, 0)),
            pl.BlockSpec((1, 8), lambda i: (0, 0)),
            pl.BlockSpec((8, 8), lambda i: (0, 0)),
            pl.BlockSpec((8, 8), lambda i: (0, 0)),
            pl.BlockSpec((1, 8), lambda i: (0, 0)),
            pl.BlockSpec((8, 1), lambda i: (0, 0)),
            pl.BlockSpec((1, 1), lambda i: (0, 0)),
            pl.BlockSpec((_BB, _F), lambda i: (i, 0)),
            pl.BlockSpec((_BB, _F), lambda i: (i, 0)),
            pl.BlockSpec((_F, 8), lambda i: (0, 0)),
        ],
        out_specs=pl.BlockSpec((_BB, 1), lambda i: (i, 0)),
        out_shape=jax.ShapeDtypeStruct((_B, 1), f32),
        compiler_params=pltpu.CompilerParams(
            dimension_semantics=("arbitrary",),
        ),
        interpret=interpret,
    )(stm2, ftb8, l1aT, l1bT, l1b2, l2wT, l2b2,
      white_features, black_features, ftwT)
    return out
